# baseline (speedup 1.0000x reference)
import functools

import jax
import jax.numpy as jnp
from jax import lax
from jax.experimental import pallas as pl
from jax.experimental.pallas import tpu as pltpu
from jax.experimental.pallas import tpu_sc as plsc

U = 50000
I = 50000
N = U + I
E = 1600000
D = 32
N_LAYERS = 3

HALF = N // 2
HALF_PAD = 51200
N_PAD = 2 * HALF_PAD
NSC = 2
NSUB = 16
NT = NSC * NSUB
CHUNK = 128
NBUF = 5

SCAN_ROWS = -(-E // (NT * CHUNK))
PBLK = 8
SCAN_ROWS = -(-SCAN_ROWS // PBLK) * PBLK
NROWS = NT * SCAN_ROWS
E_PAD = NROWS * CHUNK
PBLKS = SCAN_ROWS // PBLK

BLK = 16
CAPROWS = 400
PROWS = 2 * NT * CAPROWS

SROWS = 50016
DUMP = HALF
OUT_PER_TILE = HALF // NSUB

_SC_PARAMS = pltpu.CompilerParams(use_tc_tiling_on_sc=False,
                                 needs_layout_passes=False)
_MESH = plsc.VectorSubcoreMesh(core_axis_name="c", subcore_axis_name="s")


def _partition_body(srcm, dstm, wm, psrc, pdst, pw, cnts,
                    src_blk, dst_blk, w_blk,
                    st_src0, st_dst0, st_w0, st_src1, st_dst1, st_w1,
                    cbuf):
    c = lax.axis_index("c")
    s = lax.axis_index("s")
    t = c * NSUB + s
    row0 = t * SCAN_ROWS
    rb0 = t * CAPROWS
    rb1 = (NT + t) * CAPROWS
    lanes = jnp.arange(16, dtype=jnp.int32)

    sts = ((st_src0, st_dst0, st_w0, rb0), (st_src1, st_dst1, st_w1, rb1))

    def flush(h, nf):
        st_s, st_d, st_w, rbase = sts[h]
        pltpu.sync_copy(st_s.at[pl.ds(0, CHUNK)], psrc.at[rbase + nf])
        pltpu.sync_copy(st_d.at[pl.ds(0, CHUNK)], pdst.at[rbase + nf])
        pltpu.sync_copy(st_w.at[pl.ds(0, CHUNK)], pw.at[rbase + nf])
        st_s[pl.ds(0, 16)] = st_s[pl.ds(CHUNK, 16)]
        st_d[pl.ds(0, 16)] = st_d[pl.ds(CHUNK, 16)]
        st_w[pl.ds(0, 16)] = st_w[pl.ds(CHUNK, 16)]

    def block(b, carry):
        off0, nf0, off1, nf1 = carry
        r = row0 + b * PBLK
        pltpu.sync_copy(srcm.at[pl.ds(r, PBLK)], src_blk)
        pltpu.sync_copy(dstm.at[pl.ds(r, PBLK)], dst_blk)
        pltpu.sync_copy(wm.at[pl.ds(r, PBLK)], w_blk)

        def row(i, carry2):
            off0, nf0, off1, nf1 = carry2
            for cc in range(CHUNK // 16):
                sv = src_blk[i, pl.ds(cc * 16, 16)]
                dv = dst_blk[i, pl.ds(cc * 16, 16)]
                wv = w_blk[i, pl.ds(cc * 16, 16)]
                m0 = dv < HALF
                n0 = plsc.all_reduce_population_count(m0)[0]
                plsc.store_compressed(st_src0.at[pl.ds(off0, 16)], sv, mask=m0)
                plsc.store_compressed(st_dst0.at[pl.ds(off0, 16)], dv, mask=m0)
                plsc.store_compressed(st_w0.at[pl.ds(off0, 16)], wv, mask=m0)
                off0 = off0 + n0
                m1 = jnp.logical_not(m0)
                plsc.store_compressed(st_src1.at[pl.ds(off1, 16)], sv, mask=m1)
                plsc.store_compressed(st_dst1.at[pl.ds(off1, 16)], dv - HALF,
                                      mask=m1)
                plsc.store_compressed(st_w1.at[pl.ds(off1, 16)], wv, mask=m1)
                off1 = off1 + (16 - n0)
                f0 = off0 >= CHUNK
                pl.when(f0)(lambda: flush(0, nf0))
                nf0 = jnp.where(f0, nf0 + 1, nf0)
                off0 = jnp.where(f0, off0 - CHUNK, off0)
                f1 = off1 >= CHUNK
                pl.when(f1)(lambda: flush(1, nf1))
                nf1 = jnp.where(f1, nf1 + 1, nf1)
                off1 = jnp.where(f1, off1 - CHUNK, off1)
            return (off0, nf0, off1, nf1)

        return lax.fori_loop(0, PBLK, row, (off0, nf0, off1, nf1))

    z = jnp.int32(0)
    off0, nf0, off1, nf1 = lax.fori_loop(0, PBLKS, block, (z, z, z, z))

    def finish(h, off, nf):
        st_s, st_d, st_w, _ = sts[h]
        for g in range(CHUNK // 16):
            m = (g * 16 + lanes) < off
            st_s[pl.ds(g * 16, 16)] = jnp.where(m, st_s[pl.ds(g * 16, 16)], 0)
            st_d[pl.ds(g * 16, 16)] = jnp.where(
                m, st_d[pl.ds(g * 16, 16)], DUMP)
            st_w[pl.ds(g * 16, 16)] = jnp.where(
                m, st_w[pl.ds(g * 16, 16)], 0.0)
        pl.when(off > 0)(lambda: flush(h, nf))
        return jnp.where(off > 0, nf + 1, nf)

    nch0 = finish(0, off0, nf0)
    nch1 = finish(1, off1, nf1)

    cbuf[pl.ds(0, 16)] = jnp.full((16,), nch0, jnp.int32)
    pltpu.sync_copy(cbuf.at[pl.ds(0, 16)], cnts.at[t])
    cbuf[pl.ds(0, 16)] = jnp.full((16,), nch1, jnp.int32)
    pltpu.sync_copy(cbuf.at[pl.ds(0, 16)], cnts.at[NT + t])


_partition = functools.partial(
    pl.kernel,
    out_type=(
        jax.ShapeDtypeStruct((PROWS, CHUNK), jnp.int32),
        jax.ShapeDtypeStruct((PROWS, CHUNK), jnp.int32),
        jax.ShapeDtypeStruct((PROWS, CHUNK), jnp.float32),
        jax.ShapeDtypeStruct((2 * NT, 16), jnp.int32),
    ),
    mesh=_MESH,
    scratch_types=[
        pltpu.VMEM((PBLK, CHUNK), jnp.int32),
        pltpu.VMEM((PBLK, CHUNK), jnp.int32),
        pltpu.VMEM((PBLK, CHUNK), jnp.float32),
        pltpu.VMEM((CHUNK + 16, ), jnp.int32),
        pltpu.VMEM((CHUNK + 16, ), jnp.int32),
        pltpu.VMEM((CHUNK + 16, ), jnp.float32),
        pltpu.VMEM((CHUNK + 16, ), jnp.int32),
        pltpu.VMEM((CHUNK + 16, ), jnp.int32),
        pltpu.VMEM((CHUNK + 16, ), jnp.float32),
        pltpu.VMEM((16,), jnp.int32),
    ],
    compiler_params=_SC_PARAMS,
)(_partition_body)


def _layer_body(x_hbm, psrc, pdst, pw, cnts, out_hbm,
                sbuf, dbuf, wbuf,
                rb_a, rb_b, rb_c, rb_d, rb_e,
                cbuf, acc, *sems):
    rbs = [rb_a, rb_b, rb_c, rb_d, rb_e]
    gsems = list(sems[:NBUF])
    ssems = list(sems[NBUF:])
    c = lax.axis_index("c")
    s = lax.axis_index("s")

    def zb(r, carry):
        rb_a[r, pl.ds(0, 16)] = jnp.zeros((16,), jnp.float32)
        rb_a[r, pl.ds(16, 16)] = jnp.zeros((16,), jnp.float32)
        return carry

    lax.fori_loop(0, 128, zb, 0)

    def za(q, carry):
        pltpu.sync_copy(rb_a, acc.at[pl.ds(s * 3126 + q * 128, 128)])
        return carry

    lax.fori_loop(0, 24, za, 0)
    pltpu.sync_copy(rb_a.at[pl.ds(0, 54)],
                    acc.at[pl.ds(s * 3126 + 24 * 128, 54)])
    plsc.subcore_barrier()

    for li in range(2):
        t = li * NSUB + s
        lrow = (c * NT + t) * CAPROWS
        pltpu.sync_copy(cnts.at[c * NT + t], cbuf)
        nch = cbuf[pl.ds(0, 16)][0]
        nb = lax.div(nch + (BLK - 1), jnp.int32(BLK))

        def group(bi, carry):
            r0 = lrow + bi * BLK
            j0 = bi * BLK
            pltpu.sync_copy(psrc.at[pl.ds(r0, BLK)], sbuf)
            pltpu.sync_copy(pdst.at[pl.ds(r0, BLK)], dbuf)
            pltpu.sync_copy(pw.at[pl.ds(r0, BLK)], wbuf)
            gcp, scp = {}, {}
            for g in range(NBUF - 2):
                def start(g=g):
                    gcp[g] = pltpu.async_copy(
                        x_hbm.at[sbuf.at[g]], rbs[g], gsems[g])

                pl.when(j0 + g < nch)(start)
            for j in range(BLK):
                def work(j=j):
                    gcp[j].wait()
                    rb = rbs[j % NBUF]

                    def scale(g, carry2):
                        wv = wbuf[j, pl.ds(g * 16, 16)]
                        for kk in range(16):
                            e = g * 16 + kk
                            ws = wv[kk]
                            rb[e, pl.ds(0, 16)] = rb[e, pl.ds(0, 16)] * ws
                            rb[e, pl.ds(16, 16)] = rb[e, pl.ds(16, 16)] * ws
                        return carry2

                    lax.fori_loop(0, CHUNK // 16, scale, 0)
                    scp[j] = pltpu.async_copy(
                        rb, acc.at[dbuf.at[j]], ssems[j % NBUF], add=True)

                pl.when(j0 + j < nch)(work)
                if j + (NBUF - 2) < BLK:
                    if j >= 2:
                        pl.when(j0 + j - 2 < nch)(
                            lambda j=j: scp[j - 2].wait())

                    def nxt(j=j):
                        jn = j + (NBUF - 2)
                        gcp[jn] = pltpu.async_copy(
                            x_hbm.at[sbuf.at[jn]], rbs[jn % NBUF],
                            gsems[jn % NBUF])

                    pl.when(j0 + j + (NBUF - 2) < nch)(nxt)
            for j in range(BLK - NBUF, BLK):
                pl.when(j0 + j < nch)(lambda j=j: scp[j].wait())
            return carry

        lax.fori_loop(0, nb, group, 0)

    plsc.subcore_barrier()

    r = s * OUT_PER_TILE
    pltpu.sync_copy(acc.at[pl.ds(r, OUT_PER_TILE)],
                    out_hbm.at[pl.ds(c * HALF_PAD + r, OUT_PER_TILE)])


_layer = functools.partial(
    pl.kernel,
    out_type=jax.ShapeDtypeStruct((N_PAD, D), jnp.float32),
    mesh=_MESH,
    scratch_types=[
        pltpu.VMEM((BLK, CHUNK), jnp.int32),
        pltpu.VMEM((BLK, CHUNK), jnp.int32),
        pltpu.VMEM((BLK, CHUNK), jnp.float32),
    ] + [pltpu.VMEM((CHUNK, D), jnp.float32)] * NBUF
      + [
        pltpu.VMEM((16,), jnp.int32),
        pltpu.VMEM_SHARED((SROWS, D), jnp.float32),
    ] + [pltpu.SemaphoreType.DMA] * (2 * NBUF),
    compiler_params=_SC_PARAMS,
)(_layer_body)


def _mean_body(a, b, c, d, o):
    o[...] = (a[...] + b[...] + c[...] + d[...]) * 0.25


_MROWS = N_PAD * D // 128
_MBLK = 1024


def _mean4(x0, x1, x2, x3):
    spec = pl.BlockSpec((_MBLK, 128), lambda g: (g, 0))
    return pl.pallas_call(
        _mean_body,
        grid=(_MROWS // _MBLK,),
        in_specs=[spec] * 4,
        out_specs=spec,
        out_shape=jax.ShapeDtypeStruct((_MROWS, 128), jnp.float32),
    )(x0.reshape(_MROWS, 128), x1.reshape(_MROWS, 128),
      x2.reshape(_MROWS, 128), x3.reshape(_MROWS, 128))


def kernel(u_emb, i_emb, edge_index, edge_weight):
    zpad = jnp.zeros((HALF_PAD - HALF, D), jnp.float32)
    x0 = jnp.concatenate([u_emb, zpad, i_emb, zpad], axis=0)
    src = edge_index[0].astype(jnp.int32)
    src = src + jnp.where(src >= HALF, HALF_PAD - HALF, 0).astype(jnp.int32)
    dst = edge_index[1].astype(jnp.int32)
    w = edge_weight.astype(jnp.float32)

    pad = E_PAD - E
    srcm = jnp.concatenate([src, jnp.zeros((pad,), jnp.int32)]).reshape(NROWS, CHUNK)
    dstm = jnp.concatenate([dst, jnp.full((pad,), N, jnp.int32)]).reshape(NROWS, CHUNK)
    wm = jnp.concatenate([w, jnp.zeros((pad,), jnp.float32)]).reshape(NROWS, CHUNK)

    psrc, pdst, pw, cnts = _partition(srcm, dstm, wm)

    xs = [x0]
    for _ in range(N_LAYERS):
        xs.append(_layer(xs[-1], psrc, pdst, pw, cnts))

    final = _mean4(*xs).reshape(N_PAD, D)
    return (final[:U], final[HALF_PAD:HALF_PAD + I])


# speedup vs baseline: 1.0750x; 1.0750x over previous
import functools

import jax
import jax.numpy as jnp
from jax import lax
from jax.experimental import pallas as pl
from jax.experimental.pallas import tpu as pltpu
from jax.experimental.pallas import tpu_sc as plsc

U = 50000
I = 50000
N = U + I
E = 1600000
D = 32
N_LAYERS = 3

HALF = N // 2
HALF_PAD = 51200
N_PAD = 2 * HALF_PAD
NSC = 2
NSUB = 16
NT = NSC * NSUB
CHUNK = 128
NBUF = 5

SCAN_ROWS = -(-E // (NT * CHUNK))
PBLK = 8
SCAN_ROWS = -(-SCAN_ROWS // PBLK) * PBLK
NROWS = NT * SCAN_ROWS
E_PAD = NROWS * CHUNK
PBLKS = SCAN_ROWS // PBLK

BLK = 16
CAPROWS = 400
PROWS = 2 * NT * CAPROWS

SROWS = 50016
DUMP = HALF
OUT_PER_TILE = HALF // NSUB

_SC_PARAMS = pltpu.CompilerParams(use_tc_tiling_on_sc=False,
                                 needs_layout_passes=False)
_MESH = plsc.VectorSubcoreMesh(core_axis_name="c", subcore_axis_name="s")


def _partition_body(srcm, dstm, wm, psrc, pdst, pw, cnts,
                    src_blk, dst_blk, w_blk,
                    st_src0, st_dst0, st_w0, st_src1, st_dst1, st_w1,
                    fb_s0, fb_d0, fb_w0, fb_s1, fb_d1, fb_w1,
                    cbuf, fsem0, fsem1):
    c = lax.axis_index("c")
    s = lax.axis_index("s")
    t = c * NSUB + s
    row0 = t * SCAN_ROWS
    rb0 = t * CAPROWS
    rb1 = (NT + t) * CAPROWS
    lanes = jnp.arange(16, dtype=jnp.int32)

    sts = ((st_src0, st_dst0, st_w0, rb0, fb_s0, fb_d0, fb_w0, fsem0),
           (st_src1, st_dst1, st_w1, rb1, fb_s1, fb_d1, fb_w1, fsem1))

    def wait_flush(h):
        st_s, st_d, st_w, rbase, fb_s, fb_d, fb_w, fsem = sts[h]
        pltpu.make_async_copy(fb_s, psrc.at[rbase], fsem).wait()
        pltpu.make_async_copy(fb_d, pdst.at[rbase], fsem).wait()
        pltpu.make_async_copy(fb_w, pw.at[rbase], fsem).wait()

    def flush(h, nf):
        st_s, st_d, st_w, rbase, fb_s, fb_d, fb_w, fsem = sts[h]
        pl.when(nf >= 1)(lambda: wait_flush(h))
        for g in range(CHUNK // 16):
            fb_s[pl.ds(g * 16, 16)] = st_s[pl.ds(g * 16, 16)]
            fb_d[pl.ds(g * 16, 16)] = st_d[pl.ds(g * 16, 16)]
            fb_w[pl.ds(g * 16, 16)] = st_w[pl.ds(g * 16, 16)]
        pltpu.async_copy(fb_s, psrc.at[rbase + nf], fsem)
        pltpu.async_copy(fb_d, pdst.at[rbase + nf], fsem)
        pltpu.async_copy(fb_w, pw.at[rbase + nf], fsem)
        st_s[pl.ds(0, 16)] = st_s[pl.ds(CHUNK, 16)]
        st_d[pl.ds(0, 16)] = st_d[pl.ds(CHUNK, 16)]
        st_w[pl.ds(0, 16)] = st_w[pl.ds(CHUNK, 16)]

    def block(b, carry):
        off0, nf0, off1, nf1 = carry
        r = row0 + b * PBLK
        pltpu.sync_copy(srcm.at[pl.ds(r, PBLK)], src_blk)
        pltpu.sync_copy(dstm.at[pl.ds(r, PBLK)], dst_blk)
        pltpu.sync_copy(wm.at[pl.ds(r, PBLK)], w_blk)

        def row(i, carry2):
            off0, nf0, off1, nf1 = carry2
            for cc in range(CHUNK // 16):
                sv = src_blk[i, pl.ds(cc * 16, 16)]
                dv = dst_blk[i, pl.ds(cc * 16, 16)]
                wv = w_blk[i, pl.ds(cc * 16, 16)]
                m0 = dv < HALF
                n0 = plsc.all_reduce_population_count(m0)[0]
                plsc.store_compressed(st_src0.at[pl.ds(off0, 16)], sv, mask=m0)
                plsc.store_compressed(st_dst0.at[pl.ds(off0, 16)], dv, mask=m0)
                plsc.store_compressed(st_w0.at[pl.ds(off0, 16)], wv, mask=m0)
                off0 = off0 + n0
                m1 = jnp.logical_not(m0)
                plsc.store_compressed(st_src1.at[pl.ds(off1, 16)], sv, mask=m1)
                plsc.store_compressed(st_dst1.at[pl.ds(off1, 16)], dv - HALF,
                                      mask=m1)
                plsc.store_compressed(st_w1.at[pl.ds(off1, 16)], wv, mask=m1)
                off1 = off1 + (16 - n0)
                f0 = off0 >= CHUNK
                pl.when(f0)(lambda: flush(0, nf0))
                nf0 = jnp.where(f0, nf0 + 1, nf0)
                off0 = jnp.where(f0, off0 - CHUNK, off0)
                f1 = off1 >= CHUNK
                pl.when(f1)(lambda: flush(1, nf1))
                nf1 = jnp.where(f1, nf1 + 1, nf1)
                off1 = jnp.where(f1, off1 - CHUNK, off1)
            return (off0, nf0, off1, nf1)

        return lax.fori_loop(0, PBLK, row, (off0, nf0, off1, nf1))

    z = jnp.int32(0)
    off0, nf0, off1, nf1 = lax.fori_loop(0, PBLKS, block, (z, z, z, z))

    def finish(h, off, nf):
        st_s, st_d, st_w = sts[h][:3]
        for g in range(CHUNK // 16):
            m = (g * 16 + lanes) < off
            st_s[pl.ds(g * 16, 16)] = jnp.where(m, st_s[pl.ds(g * 16, 16)], 0)
            st_d[pl.ds(g * 16, 16)] = jnp.where(
                m, st_d[pl.ds(g * 16, 16)], DUMP)
            st_w[pl.ds(g * 16, 16)] = jnp.where(
                m, st_w[pl.ds(g * 16, 16)], 0.0)
        pl.when(off > 0)(lambda: flush(h, nf))
        nch = jnp.where(off > 0, nf + 1, nf)
        pl.when(nch >= 1)(lambda: wait_flush(h))
        return nch

    nch0 = finish(0, off0, nf0)
    nch1 = finish(1, off1, nf1)

    cbuf[pl.ds(0, 16)] = jnp.full((16,), nch0, jnp.int32)
    pltpu.sync_copy(cbuf.at[pl.ds(0, 16)], cnts.at[t])
    cbuf[pl.ds(0, 16)] = jnp.full((16,), nch1, jnp.int32)
    pltpu.sync_copy(cbuf.at[pl.ds(0, 16)], cnts.at[NT + t])


_partition = functools.partial(
    pl.kernel,
    out_type=(
        jax.ShapeDtypeStruct((PROWS, CHUNK), jnp.int32),
        jax.ShapeDtypeStruct((PROWS, CHUNK), jnp.int32),
        jax.ShapeDtypeStruct((PROWS, CHUNK), jnp.float32),
        jax.ShapeDtypeStruct((2 * NT, 16), jnp.int32),
    ),
    mesh=_MESH,
    scratch_types=[
        pltpu.VMEM((PBLK, CHUNK), jnp.int32),
        pltpu.VMEM((PBLK, CHUNK), jnp.int32),
        pltpu.VMEM((PBLK, CHUNK), jnp.float32),
        pltpu.VMEM((CHUNK + 16, ), jnp.int32),
        pltpu.VMEM((CHUNK + 16, ), jnp.int32),
        pltpu.VMEM((CHUNK + 16, ), jnp.float32),
        pltpu.VMEM((CHUNK + 16, ), jnp.int32),
        pltpu.VMEM((CHUNK + 16, ), jnp.int32),
        pltpu.VMEM((CHUNK + 16, ), jnp.float32),
        pltpu.VMEM((CHUNK,), jnp.int32),
        pltpu.VMEM((CHUNK,), jnp.int32),
        pltpu.VMEM((CHUNK,), jnp.float32),
        pltpu.VMEM((CHUNK,), jnp.int32),
        pltpu.VMEM((CHUNK,), jnp.int32),
        pltpu.VMEM((CHUNK,), jnp.float32),
        pltpu.VMEM((16,), jnp.int32),
        pltpu.SemaphoreType.DMA,
        pltpu.SemaphoreType.DMA,
    ],
    compiler_params=_SC_PARAMS,
)(_partition_body)


def _layer_body(x_hbm, psrc, pdst, pw, cnts, out_hbm,
                sbuf, dbuf, wbuf,
                rb_a, rb_b, rb_c, rb_d, rb_e,
                cbuf, acc, *sems):
    rbs = [rb_a, rb_b, rb_c, rb_d, rb_e]
    gsems = list(sems[:NBUF])
    ssems = list(sems[NBUF:])
    c = lax.axis_index("c")
    s = lax.axis_index("s")

    def zb(r, carry):
        rb_a[r, pl.ds(0, 16)] = jnp.zeros((16,), jnp.float32)
        rb_a[r, pl.ds(16, 16)] = jnp.zeros((16,), jnp.float32)
        return carry

    lax.fori_loop(0, 128, zb, 0)

    def za(q, carry):
        pltpu.sync_copy(rb_a, acc.at[pl.ds(s * 3126 + q * 128, 128)])
        return carry

    lax.fori_loop(0, 24, za, 0)
    pltpu.sync_copy(rb_a.at[pl.ds(0, 54)],
                    acc.at[pl.ds(s * 3126 + 24 * 128, 54)])
    plsc.subcore_barrier()

    for li in range(2):
        t = li * NSUB + s
        lrow = (c * NT + t) * CAPROWS
        pltpu.sync_copy(cnts.at[c * NT + t], cbuf)
        nch = cbuf[pl.ds(0, 16)][0]
        nb = lax.div(nch + (BLK - 1), jnp.int32(BLK))

        def group(bi, carry):
            r0 = lrow + bi * BLK
            j0 = bi * BLK
            pltpu.sync_copy(psrc.at[pl.ds(r0, BLK)], sbuf)
            pltpu.sync_copy(pdst.at[pl.ds(r0, BLK)], dbuf)
            pltpu.sync_copy(pw.at[pl.ds(r0, BLK)], wbuf)
            gcp, scp = {}, {}
            for g in range(NBUF - 2):
                def start(g=g):
                    gcp[g] = pltpu.async_copy(
                        x_hbm.at[sbuf.at[g]], rbs[g], gsems[g])

                pl.when(j0 + g < nch)(start)
            for j in range(BLK):
                def work(j=j):
                    gcp[j].wait()
                    rb = rbs[j % NBUF]

                    def scale(g, carry2):
                        wv = wbuf[j, pl.ds(g * 16, 16)]
                        for kk in range(16):
                            e = g * 16 + kk
                            ws = wv[kk]
                            rb[e, pl.ds(0, 16)] = rb[e, pl.ds(0, 16)] * ws
                            rb[e, pl.ds(16, 16)] = rb[e, pl.ds(16, 16)] * ws
                        return carry2

                    lax.fori_loop(0, CHUNK // 16, scale, 0)
                    scp[j] = pltpu.async_copy(
                        rb, acc.at[dbuf.at[j]], ssems[j % NBUF], add=True)

                pl.when(j0 + j < nch)(work)
                if j + (NBUF - 2) < BLK:
                    if j >= 2:
                        pl.when(j0 + j - 2 < nch)(
                            lambda j=j: scp[j - 2].wait())

                    def nxt(j=j):
                        jn = j + (NBUF - 2)
                        gcp[jn] = pltpu.async_copy(
                            x_hbm.at[sbuf.at[jn]], rbs[jn % NBUF],
                            gsems[jn % NBUF])

                    pl.when(j0 + j + (NBUF - 2) < nch)(nxt)
            for j in range(BLK - NBUF, BLK):
                pl.when(j0 + j < nch)(lambda j=j: scp[j].wait())
            return carry

        lax.fori_loop(0, nb, group, 0)

    plsc.subcore_barrier()

    r = s * OUT_PER_TILE
    pltpu.sync_copy(acc.at[pl.ds(r, OUT_PER_TILE)],
                    out_hbm.at[pl.ds(c * HALF_PAD + r, OUT_PER_TILE)])


_layer = functools.partial(
    pl.kernel,
    out_type=jax.ShapeDtypeStruct((N_PAD, D), jnp.float32),
    mesh=_MESH,
    scratch_types=[
        pltpu.VMEM((BLK, CHUNK), jnp.int32),
        pltpu.VMEM((BLK, CHUNK), jnp.int32),
        pltpu.VMEM((BLK, CHUNK), jnp.float32),
    ] + [pltpu.VMEM((CHUNK, D), jnp.float32)] * NBUF
      + [
        pltpu.VMEM((16,), jnp.int32),
        pltpu.VMEM_SHARED((SROWS, D), jnp.float32),
    ] + [pltpu.SemaphoreType.DMA] * (2 * NBUF),
    compiler_params=_SC_PARAMS,
)(_layer_body)


def _mean_body(a, b, c, d, o):
    o[...] = (a[...] + b[...] + c[...] + d[...]) * 0.25


_MROWS = N_PAD * D // 128
_MBLK = 1024


def _mean4(x0, x1, x2, x3):
    spec = pl.BlockSpec((_MBLK, 128), lambda g: (g, 0))
    return pl.pallas_call(
        _mean_body,
        grid=(_MROWS // _MBLK,),
        in_specs=[spec] * 4,
        out_specs=spec,
        out_shape=jax.ShapeDtypeStruct((_MROWS, 128), jnp.float32),
    )(x0.reshape(_MROWS, 128), x1.reshape(_MROWS, 128),
      x2.reshape(_MROWS, 128), x3.reshape(_MROWS, 128))


def kernel(u_emb, i_emb, edge_index, edge_weight):
    zpad = jnp.zeros((HALF_PAD - HALF, D), jnp.float32)
    x0 = jnp.concatenate([u_emb, zpad, i_emb, zpad], axis=0)
    src = edge_index[0].astype(jnp.int32)
    src = src + jnp.where(src >= HALF, HALF_PAD - HALF, 0).astype(jnp.int32)
    dst = edge_index[1].astype(jnp.int32)
    w = edge_weight.astype(jnp.float32)

    pad = E_PAD - E
    srcm = jnp.concatenate([src, jnp.zeros((pad,), jnp.int32)]).reshape(NROWS, CHUNK)
    dstm = jnp.concatenate([dst, jnp.full((pad,), N, jnp.int32)]).reshape(NROWS, CHUNK)
    wm = jnp.concatenate([w, jnp.zeros((pad,), jnp.float32)]).reshape(NROWS, CHUNK)

    psrc, pdst, pw, cnts = _partition(srcm, dstm, wm)

    xs = [x0]
    for _ in range(N_LAYERS):
        xs.append(_layer(xs[-1], psrc, pdst, pw, cnts))

    final = _mean4(*xs).reshape(N_PAD, D)
    return (final[:U], final[HALF_PAD:HALF_PAD + I])
